# baseline (speedup 1.0000x reference)
import jax
import jax.numpy as jnp
from jax.experimental import pallas as pl
from jax.experimental.pallas import tpu as pltpu

V_BLK = 2048


def _make_body(h_ref):
    def _body(w_ref, o_ref):
        o_ref[:, 0, :] = jax.lax.dot_general(
            h_ref[:, 0, :], w_ref[...],
            dimension_numbers=(((1,), (1,)), ((), ())),
            preferred_element_type=jnp.float32,
        )
    return _body


def _outer(h_ref, w_hbm, o_hbm):
    b, s, d = h_ref.shape
    nblk = pl.cdiv(o_hbm.shape[2], V_BLK)
    pipeline = pltpu.emit_pipeline(
        _make_body(h_ref),
        grid=(nblk,),
        in_specs=[
            pl.BlockSpec((V_BLK, d), lambda i: (i, 0),
                         pipeline_mode=pl.Buffered(buffer_count=4)),
        ],
        out_specs=[
            pl.BlockSpec((b, s, V_BLK), lambda i: (0, 0, i)),
        ],
    )
    pipeline(w_hbm, o_hbm)


def kernel(hidden_states, weight):
    b, s, d = hidden_states.shape
    v = weight.shape[0]
    return pl.pallas_call(
        _outer,
        in_specs=[
            pl.BlockSpec(memory_space=pltpu.VMEM),
            pl.BlockSpec(memory_space=pl.ANY),
        ],
        out_specs=pl.BlockSpec(memory_space=pl.ANY),
        out_shape=jax.ShapeDtypeStruct((b, s, v), jnp.float32),
    )(hidden_states, weight)


# speedup vs baseline: 1.0123x; 1.0123x over previous
import jax
import jax.numpy as jnp
from jax.experimental import pallas as pl
from jax.experimental.pallas import tpu as pltpu

V_BLK = 1024


def _make_body(h_ref):
    def _body(w_ref, o_ref):
        o_ref[:, 0, :] = jax.lax.dot_general(
            h_ref[:, 0, :], w_ref[...],
            dimension_numbers=(((1,), (1,)), ((), ())),
            preferred_element_type=jnp.float32,
        )
    return _body


def _outer(h_ref, w_hbm, o_hbm):
    b, s, d = h_ref.shape
    nblk = pl.cdiv(o_hbm.shape[2], V_BLK)
    pipeline = pltpu.emit_pipeline(
        _make_body(h_ref),
        grid=(nblk,),
        in_specs=[
            pl.BlockSpec((V_BLK, d), lambda i: (i, 0),
                         pipeline_mode=pl.Buffered(buffer_count=8)),
        ],
        out_specs=[
            pl.BlockSpec((b, s, V_BLK), lambda i: (0, 0, i)),
        ],
    )
    pipeline(w_hbm, o_hbm)


def kernel(hidden_states, weight):
    b, s, d = hidden_states.shape
    v = weight.shape[0]
    return pl.pallas_call(
        _outer,
        in_specs=[
            pl.BlockSpec(memory_space=pltpu.VMEM),
            pl.BlockSpec(memory_space=pl.ANY),
        ],
        out_specs=pl.BlockSpec(memory_space=pl.ANY),
        out_shape=jax.ShapeDtypeStruct((b, s, v), jnp.float32),
    )(hidden_states, weight)


# speedup vs baseline: 1.0154x; 1.0031x over previous
import jax
import jax.numpy as jnp
from jax.experimental import pallas as pl
from jax.experimental.pallas import tpu as pltpu

V_BLK = 2048


def _head_kernel(h_ref, w_ref, o_ref):
    o_ref[:, 0, :] = jax.lax.dot_general(
        h_ref[:, 0, :], w_ref[...],
        dimension_numbers=(((1,), (1,)), ((), ())),
        preferred_element_type=jnp.float32,
    )


def kernel(hidden_states, weight):
    b, s, d = hidden_states.shape
    v = weight.shape[0]
    nblk = pl.cdiv(v, V_BLK)
    out = pl.pallas_call(
        _head_kernel,
        grid=(nblk,),
        in_specs=[
            pl.BlockSpec((b, s, d), lambda i: (0, 0, 0)),
            pl.BlockSpec((V_BLK, d), lambda i: (i, 0)),
        ],
        out_specs=pl.BlockSpec((b, s, V_BLK), lambda i: (0, 0, i)),
        out_shape=jax.ShapeDtypeStruct((b, s, v), jnp.float32),
        compiler_params=pltpu.CompilerParams(
            dimension_semantics=("arbitrary",),
        ),
    )(hidden_states, weight)
    return out
